# baseline (speedup 1.0000x reference)
import functools

import jax
import jax.numpy as jnp
from jax import lax
from jax.experimental import pallas as pl
from jax.experimental.pallas import tpu as pltpu
from jax.experimental.pallas import tpu_sc as plsc

_N_NODES = 10000
_N_EDGES = 320000
_D = 128
_N_GRAPHS = 8

_NC = 2
_NS = 16
_E_CHUNK = 128
_ROWS_PT = 624
_ZCH = 104
_NB = 3


def _sc_edge_aggregate(x, edge_index):
    n_tiles = _NC * _NS
    n_chunks = 78
    edges_per_tile = n_chunks * _E_CHUNK
    extra0 = n_tiles * edges_per_tile
    tail0 = _NS * _ROWS_PT
    tail_rows = _N_NODES - tail0

    mesh = plsc.VectorSubcoreMesh(core_axis_name="c", subcore_axis_name="s")

    @functools.partial(
        pl.kernel,
        mesh=mesh,
        out_type=jax.ShapeDtypeStruct((_NC, _N_NODES, _D), jnp.float32),
        scratch_types=(
            [pltpu.VMEM((_E_CHUNK,), jnp.int32) for _ in range(_NB)]
            + [pltpu.VMEM((1, _E_CHUNK), jnp.int32) for _ in range(_NB)]
            + [pltpu.VMEM((_E_CHUNK, _D), jnp.float32) for _ in range(_NB)]
            + [pltpu.VMEM_SHARED((_N_NODES, _D), jnp.float32)]
            + [pltpu.SemaphoreType.DMA for _ in range(4 * _NB)]
        ),
    )
    def k(x_hbm, ei_hbm, out_hbm, *bufs):
        src_v = bufs[0:_NB]
        dst_v = bufs[_NB:2 * _NB]
        rows_v = bufs[2 * _NB:3 * _NB]
        agg_sh = bufs[3 * _NB]
        sems = bufs[3 * _NB + 1:]
        gsem = sems[0:_NB]
        ssem = sems[_NB:2 * _NB]
        dsem = sems[2 * _NB:3 * _NB]
        asem = sems[3 * _NB:4 * _NB]

        c = lax.axis_index("c")
        s = lax.axis_index("s")
        row0 = s * _ROWS_PT
        is_last = s == _NS - 1

        def zero_row(i, carry):
            for j in range(_D // 16):
                rows_v[0][i, pl.ds(j * 16, 16)] = jnp.zeros((16,), jnp.float32)
            return carry

        lax.fori_loop(0, _ZCH, zero_row, 0)
        for z in range(_ROWS_PT // _ZCH):
            pltpu.sync_copy(rows_v[0].at[pl.ds(0, _ZCH)],
                            agg_sh.at[pl.ds(row0 + z * _ZCH, _ZCH)])

        @pl.when(is_last)
        def _():
            pltpu.sync_copy(rows_v[0].at[pl.ds(0, tail_rows)],
                            agg_sh.at[pl.ds(tail0, tail_rows)])

        plsc.subcore_barrier()

        base_e = (c * _NS + s) * edges_per_tile

        def e0(j):
            return base_e + j * _E_CHUNK

        def issue_src(j, r):
            pltpu.async_copy(ei_hbm.at[0, pl.ds(e0(j), _E_CHUNK)],
                             src_v[r], ssem[r])

        def issue_dst(j, r):
            pltpu.async_copy(ei_hbm.at[1, pl.ds(e0(j), _E_CHUNK)],
                             dst_v[r].at[0], dsem[r])

        _H = _E_CHUNK // 2

        def issue_gather(r):
            pltpu.async_copy(x_hbm.at[src_v[r].at[pl.ds(0, _H)]],
                             rows_v[r].at[pl.ds(0, _H)], gsem[r])
            pltpu.async_copy(x_hbm.at[src_v[r].at[pl.ds(_H, _H)]],
                             rows_v[r].at[pl.ds(_H, _H)], gsem[r])

        def issue_scat(r):
            pltpu.async_copy(rows_v[r].at[pl.ds(0, _H)],
                             agg_sh.at[dst_v[r].at[0, pl.ds(0, _H)]],
                             asem[r], add=True)
            pltpu.async_copy(rows_v[r].at[pl.ds(_H, _H)],
                             agg_sh.at[dst_v[r].at[0, pl.ds(_H, _H)]],
                             asem[r], add=True)

        def wait_gather(r):
            for h in range(2):
                pltpu.make_async_copy(x_hbm.at[pl.ds(0, _H)],
                                      rows_v[r].at[pl.ds(h * _H, _H)],
                                      gsem[r]).wait()

        def wait_src(r):
            pltpu.make_async_copy(ei_hbm.at[0, pl.ds(0, _E_CHUNK)], src_v[r],
                                  ssem[r]).wait()

        def wait_dst(r):
            pltpu.make_async_copy(ei_hbm.at[1, pl.ds(0, _E_CHUNK)],
                                  dst_v[r].at[0], dsem[r]).wait()

        def wait_scat(r):
            for h in range(2):
                pltpu.make_async_copy(
                    rows_v[r].at[pl.ds(h * _H, _H)],
                    agg_sh.at[dst_v[r].at[0, pl.ds(h * _H, _H)]],
                    asem[r]).wait()

        pltpu.sync_copy(ei_hbm.at[0, pl.ds(e0(0), _E_CHUNK)], src_v[0])
        issue_gather(0)
        issue_src(1, 1)
        issue_src(2, 2)
        issue_dst(0, 0)
        issue_dst(1, 1)
        wait_src(1)
        issue_gather(1)

        def step(j, r, first=False, issue2=True, issue3=True):
            r2 = (r + 2) % _NB
            if not first:
                wait_scat(r2)
            if issue2:
                wait_src(r2)
                issue_gather(r2)
                issue_dst(j + 2, r2)
            wait_gather(r)
            wait_dst(r)
            issue_scat(r)
            if issue3:
                issue_src(j + 3, r)

        step(0, 0, first=True)
        step(1, 1)
        step(2, 2)

        def triple(t, carry):
            j = t * 3 + 3
            step(j, 0)
            step(j + 1, 1)
            step(j + 2, 2)
            return carry

        lax.fori_loop(0, 24, triple, 0)
        step(75, 0, issue3=False)
        step(76, 1, issue2=False, issue3=False)
        step(77, 2, issue2=False, issue3=False)
        wait_scat(2)

        @pl.when(s < 2)
        def _():
            ex = extra0 + (c * 2 + s) * _E_CHUNK
            pltpu.sync_copy(ei_hbm.at[0, pl.ds(ex, _E_CHUNK)], src_v[0])
            pltpu.sync_copy(ei_hbm.at[1, pl.ds(ex, _E_CHUNK)], dst_v[0].at[0])
            pltpu.async_copy(x_hbm.at[src_v[0]], rows_v[0], gsem[0]).wait()
            pltpu.sync_copy(rows_v[0], agg_sh.at[dst_v[0].at[0]], add=True)

        plsc.subcore_barrier()

        for z in range(_ROWS_PT // _ZCH):
            r = row0 + z * _ZCH
            pltpu.async_copy(agg_sh.at[pl.ds(r, _ZCH)],
                             out_hbm.at[c, pl.ds(r, _ZCH)], gsem[0])

        @pl.when(is_last)
        def _():
            pltpu.sync_copy(agg_sh.at[pl.ds(tail0, tail_rows)],
                            out_hbm.at[c, pl.ds(tail0, tail_rows)])

        for z in range(_ROWS_PT // _ZCH):
            pltpu.make_async_copy(agg_sh.at[pl.ds(row0 + z * _ZCH, _ZCH)],
                                  out_hbm.at[c, pl.ds(row0 + z * _ZCH, _ZCH)],
                                  gsem[0]).wait()

    return k(x, edge_index)


def _tc_dense(agg2, x, w, batch2d):
    def body(agg_ref, x_ref, w_ref, b_ref, out_ref):
        a = agg_ref[0] + agg_ref[1] + x_ref[...]
        h = jnp.maximum(
            jnp.dot(a, w_ref[...], preferred_element_type=jnp.float32), 0.0)
        gids = lax.broadcasted_iota(jnp.int32, (1, _N_GRAPHS), 1)
        oh = (b_ref[...].reshape(_N_NODES, 1) == gids).astype(jnp.float32)
        sums = lax.dot_general(oh, h, (((0,), (0,)), ((), ())),
                               preferred_element_type=jnp.float32)
        counts = jnp.sum(oh, axis=0)[:, None]
        gmean = sums / jnp.maximum(counts, 1.0)
        out_ref[...] = h + jnp.dot(oh, gmean,
                                   preferred_element_type=jnp.float32)

    return pl.pallas_call(
        body,
        out_shape=jax.ShapeDtypeStruct((_N_NODES, _D), jnp.float32),
    )(agg2, x, w, batch2d)


def kernel(x, edge_index, batch, W):
    agg2 = _sc_edge_aggregate(x, edge_index.astype(jnp.int32))
    return _tc_dense(agg2, x, W, batch.astype(jnp.int32))


# speedup vs baseline: 1.0138x; 1.0138x over previous
import functools

import jax
import jax.numpy as jnp
from jax import lax
from jax.experimental import pallas as pl
from jax.experimental.pallas import tpu as pltpu
from jax.experimental.pallas import tpu_sc as plsc

_N_NODES = 10000
_N_EDGES = 320000
_D = 128
_N_GRAPHS = 8

_NC = 2
_NS = 16
_E_CHUNK = 128
_ROWS_PT = 624
_ZCH = 104
_NB = 3


def _sc_edge_aggregate(x, edge_index):
    n_tiles = _NC * _NS
    n_chunks = 78
    edges_per_tile = n_chunks * _E_CHUNK
    extra0 = n_tiles * edges_per_tile
    tail0 = _NS * _ROWS_PT
    tail_rows = _N_NODES - tail0

    mesh = plsc.VectorSubcoreMesh(core_axis_name="c", subcore_axis_name="s")

    @functools.partial(
        pl.kernel,
        mesh=mesh,
        out_type=jax.ShapeDtypeStruct((_NC, _N_NODES, _D), jnp.float32),
        scratch_types=(
            [pltpu.VMEM((_E_CHUNK,), jnp.int32) for _ in range(_NB)]
            + [pltpu.VMEM((1, _E_CHUNK), jnp.int32) for _ in range(_NB)]
            + [pltpu.VMEM((_E_CHUNK, _D), jnp.float32) for _ in range(_NB)]
            + [pltpu.VMEM_SHARED((_N_NODES, _D), jnp.float32)]
            + [pltpu.SemaphoreType.DMA for _ in range(4 * _NB)]
        ),
    )
    def k(x_hbm, ei_hbm, out_hbm, *bufs):
        src_v = bufs[0:_NB]
        dst_v = bufs[_NB:2 * _NB]
        rows_v = bufs[2 * _NB:3 * _NB]
        agg_sh = bufs[3 * _NB]
        sems = bufs[3 * _NB + 1:]
        gsem = sems[0:_NB]
        ssem = sems[_NB:2 * _NB]
        dsem = sems[2 * _NB:3 * _NB]
        asem = sems[3 * _NB:4 * _NB]

        c = lax.axis_index("c")
        s = lax.axis_index("s")
        row0 = s * _ROWS_PT
        is_last = s == _NS - 1

        base_e = (c * _NS + s) * edges_per_tile

        def e0(j):
            return base_e + j * _E_CHUNK

        def issue_src(j, r):
            pltpu.async_copy(ei_hbm.at[0, pl.ds(e0(j), _E_CHUNK)],
                             src_v[r], ssem[r])

        def issue_dst(j, r):
            pltpu.async_copy(ei_hbm.at[1, pl.ds(e0(j), _E_CHUNK)],
                             dst_v[r].at[0], dsem[r])

        def issue_gather(r):
            pltpu.async_copy(x_hbm.at[src_v[r]], rows_v[r], gsem[r])

        def issue_scat(r):
            pltpu.async_copy(rows_v[r], agg_sh.at[dst_v[r].at[0]],
                             asem[r], add=True)

        def wait_gather(r):
            pltpu.make_async_copy(x_hbm.at[pl.ds(0, _E_CHUNK)], rows_v[r],
                                  gsem[r]).wait()

        def wait_src(r):
            pltpu.make_async_copy(ei_hbm.at[0, pl.ds(0, _E_CHUNK)], src_v[r],
                                  ssem[r]).wait()

        def wait_dst(r):
            pltpu.make_async_copy(ei_hbm.at[1, pl.ds(0, _E_CHUNK)],
                                  dst_v[r].at[0], dsem[r]).wait()

        def wait_scat(r):
            pltpu.make_async_copy(rows_v[r], agg_sh.at[dst_v[r].at[0]],
                                  asem[r]).wait()

        pltpu.sync_copy(ei_hbm.at[0, pl.ds(e0(0), _E_CHUNK)], src_v[0])
        issue_gather(0)
        issue_src(1, 1)
        issue_src(2, 2)
        issue_dst(0, 0)
        issue_dst(1, 1)
        wait_src(1)
        issue_gather(1)

        def zero_row(i, carry):
            for j in range(_D // 16):
                rows_v[2][i, pl.ds(j * 16, 16)] = jnp.zeros((16,), jnp.float32)
            return carry

        lax.fori_loop(0, _ZCH, zero_row, 0)
        for z in range(_ROWS_PT // _ZCH):
            pltpu.sync_copy(rows_v[2].at[pl.ds(0, _ZCH)],
                            agg_sh.at[pl.ds(row0 + z * _ZCH, _ZCH)])

        @pl.when(is_last)
        def _():
            pltpu.sync_copy(rows_v[2].at[pl.ds(0, tail_rows)],
                            agg_sh.at[pl.ds(tail0, tail_rows)])

        plsc.subcore_barrier()

        def step(j, r, first=False, issue2=True, issue3=True):
            r2 = (r + 2) % _NB
            if not first:
                wait_scat(r2)
            if issue2:
                wait_src(r2)
                issue_gather(r2)
                issue_dst(j + 2, r2)
            wait_gather(r)
            wait_dst(r)
            issue_scat(r)
            if issue3:
                issue_src(j + 3, r)

        step(0, 0, first=True)
        step(1, 1)
        step(2, 2)

        def triple(t, carry):
            j = t * 3 + 3
            step(j, 0)
            step(j + 1, 1)
            step(j + 2, 2)
            return carry

        lax.fori_loop(0, 24, triple, 0)
        step(75, 0, issue3=False)
        step(76, 1, issue2=False, issue3=False)
        step(77, 2, issue2=False, issue3=False)
        wait_scat(2)

        @pl.when(s < 2)
        def _():
            ex = extra0 + (c * 2 + s) * _E_CHUNK
            pltpu.sync_copy(ei_hbm.at[0, pl.ds(ex, _E_CHUNK)], src_v[0])
            pltpu.sync_copy(ei_hbm.at[1, pl.ds(ex, _E_CHUNK)], dst_v[0].at[0])
            pltpu.async_copy(x_hbm.at[src_v[0]], rows_v[0], gsem[0]).wait()
            pltpu.sync_copy(rows_v[0], agg_sh.at[dst_v[0].at[0]], add=True)

        plsc.subcore_barrier()

        for z in range(_ROWS_PT // _ZCH):
            r = row0 + z * _ZCH
            pltpu.async_copy(agg_sh.at[pl.ds(r, _ZCH)],
                             out_hbm.at[c, pl.ds(r, _ZCH)], gsem[0])

        @pl.when(is_last)
        def _():
            pltpu.sync_copy(agg_sh.at[pl.ds(tail0, tail_rows)],
                            out_hbm.at[c, pl.ds(tail0, tail_rows)])

        for z in range(_ROWS_PT // _ZCH):
            pltpu.make_async_copy(agg_sh.at[pl.ds(row0 + z * _ZCH, _ZCH)],
                                  out_hbm.at[c, pl.ds(row0 + z * _ZCH, _ZCH)],
                                  gsem[0]).wait()

    return k(x, edge_index)


def _tc_dense(agg2, x, w, batch2d):
    def body(agg_ref, x_ref, w_ref, b_ref, out_ref):
        a = agg_ref[0] + agg_ref[1] + x_ref[...]
        h = jnp.maximum(
            jnp.dot(a, w_ref[...], preferred_element_type=jnp.float32), 0.0)
        gids = lax.broadcasted_iota(jnp.int32, (1, _N_GRAPHS), 1)
        oh = (b_ref[...].reshape(_N_NODES, 1) == gids).astype(jnp.float32)
        sums = lax.dot_general(oh, h, (((0,), (0,)), ((), ())),
                               preferred_element_type=jnp.float32)
        counts = jnp.sum(oh, axis=0)[:, None]
        gmean = sums / jnp.maximum(counts, 1.0)
        out_ref[...] = h + jnp.dot(oh, gmean,
                                   preferred_element_type=jnp.float32)

    return pl.pallas_call(
        body,
        out_shape=jax.ShapeDtypeStruct((_N_NODES, _D), jnp.float32),
    )(agg2, x, w, batch2d)


def kernel(x, edge_index, batch, W):
    agg2 = _sc_edge_aggregate(x, edge_index.astype(jnp.int32))
    return _tc_dense(agg2, x, W, batch.astype(jnp.int32))
